# baseline (speedup 1.0000x reference)
import functools

import jax
import jax.numpy as jnp
from jax import lax
from jax.experimental import pallas as pl
from jax.experimental.pallas import tpu as pltpu
from jax.experimental.pallas import tpu_sc as plsc
from jax._src.pallas import core as pallas_core
from jax._src.pallas import mpmd

MAX_LEN = 8192
D = 1024
B_TOTAL = 4 * 8192

_info = plsc.get_sparse_core_info()
NC = _info.num_cores
NS = _info.num_subcores
NW = NC * NS
B_PER_W = B_TOTAL // NW
R = 16
N_R = B_PER_W // R
S = 3


@jax.jit
def _gather_rows(table, idx3):
  smesh = plsc.ScalarSubcoreMesh(axis_name="c", num_cores=NC)
  vmesh = plsc.VectorSubcoreMesh(core_axis_name="c", subcore_axis_name="s")

  sem_scs = pallas_core.CoreMemorySpace(pltpu.MemorySpace.SEMAPHORE, smesh)
  vmem_tec = pallas_core.CoreMemorySpace(pltpu.MemorySpace.VMEM, vmesh)
  sem_tec = pallas_core.CoreMemorySpace(pltpu.MemorySpace.SEMAPHORE, vmesh)

  scratch_types = [
      pltpu.MemorySpace.VMEM_SHARED((S, NS, R, D), jnp.float32),
      vmem_tec((N_R, R), jnp.int32),
      vmem_tec((S, R, D), jnp.float32),
      sem_tec((S,), pltpu.SemaphoreType.DMA.dtype),
      sem_tec((S,), pltpu.SemaphoreType.DMA.dtype),
      sem_scs((S,), pltpu.SemaphoreType.REGULAR.dtype),
      sem_tec((S,), pltpu.SemaphoreType.REGULAR.dtype),
      sem_scs((), pltpu.SemaphoreType.DMA.dtype),
  ]

  def tec_fn(table_hbm, idx_hbm, out_hbm, ring, idx_v, tbuf, gsem, csem,
             t2s, s2t, wsem):
    del out_hbm, wsem
    sid = lax.axis_index("s")
    wid = sid * NC + lax.axis_index("c")
    pltpu.sync_copy(idx_hbm.at[wid], idx_v)

    def start_gather(r, k):
      pltpu.async_copy(table_hbm.at[idx_v.at[r]], tbuf.at[k], gsem.at[k])

    def wait_gather(r, k):
      pltpu.make_async_copy(
          table_hbm.at[idx_v.at[r]], tbuf.at[k], gsem.at[k]).wait()

    def start_push(k):
      pltpu.async_copy(tbuf.at[k], ring.at[k, sid], csem.at[k])

    def wait_push(k):
      pltpu.make_async_copy(tbuf.at[k], ring.at[k, sid], csem.at[k]).wait()

    start_gather(0, 0)

    def round_body(r, carry):
      for k in range(S):
        @pl.when(r % S == k)
        def _():
          kp = (k + 1) % S
          @pl.when(r >= 2)
          def _():
            wait_push(kp)
            pltpu.semaphore_signal(t2s.at[kp], 1)
          @pl.when(r + 1 < N_R)
          def _():
            start_gather(r + 1, kp)
          wait_gather(r, k)
          @pl.when(r >= S)
          def _():
            pltpu.semaphore_wait(s2t.at[k], 1)
          start_push(k)
      return carry

    lax.fori_loop(0, N_R, round_body, 0, unroll=False)

    for r in (N_R - 2, N_R - 1):
      wait_push(r % S)
      pltpu.semaphore_signal(t2s.at[r % S], 1)
    for b in range(S):
      pltpu.semaphore_wait(s2t.at[b], 1)

  def scs_fn(table_hbm, idx_hbm, out_hbm, ring, idx_v, tbuf, gsem, csem,
             t2s, s2t, wsem):
    del table_hbm, idx_hbm, idx_v, tbuf, gsem, csem
    myc = lax.axis_index("c")

    def round_body(r, carry):
      for k in range(S):
        @pl.when(r % S == k)
        def _():
          pltpu.semaphore_wait(t2s.at[k], NS)
          for s in range(NS):
            pltpu.async_copy(
                ring.at[k, s],
                out_hbm.at[pl.ds((s * NC + myc) * B_PER_W + r * R, R)],
                wsem)
          for s in range(NS):
            pltpu.make_async_copy(
                ring.at[k, s],
                out_hbm.at[pl.ds((s * NC + myc) * B_PER_W + r * R, R)],
                wsem).wait()
          for s in range(NS):
            pltpu.semaphore_signal(s2t.at[k], 1, device_id={"s": s})
      return carry

    lax.fori_loop(0, N_R, round_body, 0, unroll=False)

  run = mpmd.mpmd_map(
      [(smesh, scs_fn), (vmesh, tec_fn)],
      out_types=jax.ShapeDtypeStruct((B_TOTAL, D), jnp.float32),
      scratch_types=scratch_types,
  )
  return run(table, idx3)


def kernel(inputs, positions, position_embeddings):
  idx3 = positions.reshape(NW, N_R, R).astype(jnp.int32)
  out = _gather_rows(position_embeddings, idx3)
  return out.reshape(inputs.shape)


# speedup vs baseline: 1.1347x; 1.1347x over previous
import functools

import jax
import jax.numpy as jnp
from jax import lax
from jax.experimental import pallas as pl
from jax.experimental.pallas import tpu as pltpu
from jax.experimental.pallas import tpu_sc as plsc
from jax._src.pallas import core as pallas_core
from jax._src.pallas import mpmd

MAX_LEN = 8192
D = 1024
B_TOTAL = 4 * 8192

_info = plsc.get_sparse_core_info()
NC = _info.num_cores
NS = _info.num_subcores
NW = NC * NS
B_PER_W = B_TOTAL // NW
R = 16
N_R = B_PER_W // R
S = 3


@jax.jit
def _gather_rows(table, idx3):
  smesh = plsc.ScalarSubcoreMesh(axis_name="c", num_cores=NC)
  vmesh = plsc.VectorSubcoreMesh(core_axis_name="c", subcore_axis_name="s")

  sem_scs = pallas_core.CoreMemorySpace(pltpu.MemorySpace.SEMAPHORE, smesh)
  vmem_tec = pallas_core.CoreMemorySpace(pltpu.MemorySpace.VMEM, vmesh)
  sem_tec = pallas_core.CoreMemorySpace(pltpu.MemorySpace.SEMAPHORE, vmesh)

  scratch_types = [
      pltpu.MemorySpace.VMEM_SHARED((S, NS, R, D), jnp.float32),
      vmem_tec((N_R, R), jnp.int32),
      vmem_tec((S, R, D), jnp.float32),
      sem_tec((S,), pltpu.SemaphoreType.DMA.dtype),
      sem_tec((S,), pltpu.SemaphoreType.DMA.dtype),
      sem_scs((S,), pltpu.SemaphoreType.REGULAR.dtype),
      sem_tec((S,), pltpu.SemaphoreType.REGULAR.dtype),
      sem_scs((), pltpu.SemaphoreType.DMA.dtype),
  ]

  def tec_fn(table_hbm, idx_hbm, out_hbm, ring, idx_v, tbuf, gsem, csem,
             t2s, s2t, wsem):
    del out_hbm, wsem
    sid = lax.axis_index("s")
    wid = sid * NC + lax.axis_index("c")
    pltpu.sync_copy(idx_hbm.at[wid], idx_v)

    def start_gather(r, k):
      pltpu.async_copy(table_hbm.at[idx_v.at[r]], tbuf.at[k], gsem.at[k])

    def wait_gather(r, k):
      pltpu.make_async_copy(
          table_hbm.at[idx_v.at[r]], tbuf.at[k], gsem.at[k]).wait()

    def start_push(k):
      pltpu.async_copy(tbuf.at[k], ring.at[k, sid], csem.at[k])

    def wait_push(k):
      pltpu.make_async_copy(tbuf.at[k], ring.at[k, sid], csem.at[k]).wait()

    start_gather(0, 0)

    def round_body(r, carry):
      for k in range(S):
        @pl.when(r % S == k)
        def _():
          kp = (k + 1) % S
          @pl.when(r >= 2)
          def _():
            wait_push(kp)
            pltpu.semaphore_signal(t2s.at[kp], 1)
          @pl.when(r + 1 < N_R)
          def _():
            start_gather(r + 1, kp)
          wait_gather(r, k)
          @pl.when(r >= S)
          def _():
            pltpu.semaphore_wait(s2t.at[k], 1)
          start_push(k)
      return carry

    lax.fori_loop(0, N_R, round_body, 0, unroll=False)

    for r in (N_R - 2, N_R - 1):
      wait_push(r % S)
      pltpu.semaphore_signal(t2s.at[r % S], 1)
    for b in range(S):
      pltpu.semaphore_wait(s2t.at[b], 1)

  def scs_fn(table_hbm, idx_hbm, out_hbm, ring, idx_v, tbuf, gsem, csem,
             t2s, s2t, wsem):
    del table_hbm, idx_hbm, idx_v, tbuf, gsem, csem
    myc = lax.axis_index("c")

    def issue(r, k):
      for s in range(NS):
        pltpu.async_copy(
            ring.at[k, s],
            out_hbm.at[pl.ds((s * NC + myc) * B_PER_W + r * R, R)],
            wsem)

    def drain_and_release(r, k):
      for s in range(NS):
        pltpu.make_async_copy(
            ring.at[k, s],
            out_hbm.at[pl.ds((s * NC + myc) * B_PER_W + r * R, R)],
            wsem).wait()
      for s in range(NS):
        pltpu.semaphore_signal(s2t.at[k], 1, device_id={"s": s})

    def round_body(r, carry):
      for k in range(S):
        @pl.when(r % S == k)
        def _():
          pltpu.semaphore_wait(t2s.at[k], NS)
          issue(r, k)
          @pl.when(r >= 1)
          def _():
            drain_and_release(r - 1, (k + S - 1) % S)
      return carry

    lax.fori_loop(0, N_R, round_body, 0, unroll=False)
    drain_and_release(N_R - 1, (N_R - 1) % S)

  run = mpmd.mpmd_map(
      [(smesh, scs_fn), (vmesh, tec_fn)],
      out_types=jax.ShapeDtypeStruct((B_TOTAL, D), jnp.float32),
      scratch_types=scratch_types,
  )
  return run(table, idx3)


def kernel(inputs, positions, position_embeddings):
  idx3 = positions.reshape(NW, N_R, R).astype(jnp.int32)
  out = _gather_rows(position_embeddings, idx3)
  return out.reshape(inputs.shape)


# speedup vs baseline: 1.2278x; 1.0820x over previous
import functools

import jax
import jax.numpy as jnp
from jax import lax
from jax.experimental import pallas as pl
from jax.experimental.pallas import tpu as pltpu
from jax.experimental.pallas import tpu_sc as plsc

MAX_LEN = 8192
D = 1024
B_TOTAL = 4 * 8192

_info = plsc.get_sparse_core_info()
NC = _info.num_cores
NS = _info.num_subcores
NW = NC * NS
B_PER_W = B_TOTAL // NW
CH = 32
N_CHUNKS = B_PER_W // CH
NB = 3
K = 2
N_MAIN = (N_CHUNKS // NB) * NB


@jax.jit
def _gather_rows(table, idx3):
  mesh = plsc.VectorSubcoreMesh(core_axis_name="c", subcore_axis_name="s")

  @functools.partial(
      pl.kernel,
      mesh=mesh,
      out_type=jax.ShapeDtypeStruct((B_TOTAL, D), jnp.float32),
      scratch_types=[
          pltpu.VMEM((N_CHUNKS, CH), jnp.int32),
          pltpu.VMEM((NB, CH, D), jnp.float32),
          pltpu.SemaphoreType.DMA((NB,)),
          pltpu.SemaphoreType.DMA((NB,)),
      ],
  )
  def k(table_hbm, idx_hbm, out_hbm, idx_v, rows_v, gsem, ssem):
    wid = lax.axis_index("s") * NC + lax.axis_index("c")
    base = wid * B_PER_W
    pltpu.sync_copy(idx_hbm.at[wid], idx_v)

    def start_gather(c, b):
      pltpu.async_copy(table_hbm.at[idx_v.at[c]], rows_v.at[b], gsem.at[b])

    def wait_gather(c, b):
      pltpu.make_async_copy(
          table_hbm.at[idx_v.at[c]], rows_v.at[b], gsem.at[b]).wait()

    def start_scatter(c, b):
      pltpu.async_copy(
          rows_v.at[b], out_hbm.at[pl.ds(base + c * CH, CH)], ssem.at[b])

    def wait_scatter(c, b):
      pltpu.make_async_copy(
          rows_v.at[b], out_hbm.at[pl.ds(base + c * CH, CH)], ssem.at[b]).wait()

    def step(c, b, static):
      bk = (b + K) % NB
      if static:
        if c >= NB - K:
          wait_scatter(c - (NB - K), bk)
        if c + K < N_CHUNKS:
          start_gather(c + K, bk)
      else:
        @pl.when(c >= NB - K)
        def _():
          wait_scatter(c - (NB - K), bk)
        @pl.when(c + K < N_CHUNKS)
        def _():
          start_gather(c + K, bk)
      wait_gather(c, b)
      start_scatter(c, b)

    for b in range(K):
      start_gather(b, b)

    def outer(i, carry):
      g = i * NB
      for b in range(NB):
        step(g + b, b, static=False)
      return carry

    lax.fori_loop(0, N_MAIN // NB, outer, 0, unroll=False)

    for c in range(N_MAIN, N_CHUNKS):
      step(c, c % NB, static=True)

    for c in range(N_CHUNKS - (NB - K), N_CHUNKS):
      wait_scatter(c, c % NB)

  return k(table, idx3)


def kernel(inputs, positions, position_embeddings):
  idx3 = positions.reshape(NW, N_CHUNKS, CH).astype(jnp.int32)
  out = _gather_rows(position_embeddings, idx3)
  return out.reshape(inputs.shape)
